# Initial kernel scaffold; baseline (speedup 1.0000x reference)
import jax
import jax.numpy as jnp
from jax.experimental import pallas as pl


def kernel(x, mask, emb_i_weight, emb_j_weight):
    raise NotImplementedError("write your pallas kernel here")


# speedup vs baseline: 1.0729x; 1.0729x over previous
import functools

import jax
import jax.numpy as jnp
from jax.experimental import pallas as pl
from jax.experimental.pallas import tpu as pltpu

_CHUNK = 128
_CUT = 0.05


def _lr_kernel(mcol_ref, mrow_ref, ei_ref, ej_ref, xj_ref, xi_ref, out_ref,
               accy_ref, num_ref, *, blk, batch):
    i = pl.program_id(0)
    j = pl.program_id(1)
    nj = pl.num_programs(1)

    ei = ei_ref[...]
    ej = ej_ref[...]
    ein = ei / jnp.maximum(
        jnp.sqrt(jnp.sum(ei * ei, axis=1, keepdims=True)), 1e-8)
    ejn = ej / jnp.maximum(
        jnp.sqrt(jnp.sum(ej * ej, axis=1, keepdims=True)), 1e-8)
    s = jnp.abs(jax.lax.dot_general(
        ein, ejn, (((1,), (1,)), ((), ())),
        preferred_element_type=jnp.float32))
    mi = mcol_ref[0].astype(jnp.float32)
    mj = mrow_ref[0].astype(jnp.float32)
    s = s * (mi * mj)
    ii = i * blk + jax.lax.broadcasted_iota(jnp.int32, (blk, blk), 0)
    jjp = j * blk + jax.lax.broadcasted_iota(jnp.int32, (blk, blk), 1)
    keep = (jnp.abs(ii - jjp) > _CHUNK) & (s > _CUT)
    w = jnp.where(keep, s, 0.0)
    cnt = jnp.sum(keep.astype(jnp.float32), axis=1, keepdims=True)

    @pl.when(j == 0)
    def _init():
        num_ref[...] = cnt
        for b in range(batch):
            accy_ref[b] = jnp.dot(w, xj_ref[b],
                                  preferred_element_type=jnp.float32)

    @pl.when(j > 0)
    def _acc():
        num_ref[...] += cnt
        for b in range(batch):
            accy_ref[b] += jnp.dot(w, xj_ref[b],
                                   preferred_element_type=jnp.float32)

    @pl.when(j == nj - 1)
    def _fin():
        num = num_ref[...]
        xi = xi_ref[...]
        y = accy_ref[...] / jnp.maximum(num, 1.0)[None]
        out_ref[...] = jnp.where((num > 0.0)[None], (xi + y) * 0.5, xi)


@jax.jit
def kernel(x, mask, emb_i_weight, emb_j_weight):
    B, L, D = x.shape
    E = emb_i_weight.shape[1]
    blk = 512 if L % 512 == 0 else 128
    nb = L // blk
    mask_col = mask.reshape(nb, blk, 1)
    mask_row = mask.reshape(nb, 1, blk)
    return pl.pallas_call(
        functools.partial(_lr_kernel, blk=blk, batch=B),
        grid=(nb, nb),
        in_specs=[
            pl.BlockSpec((1, blk, 1), lambda i, j: (i, 0, 0)),
            pl.BlockSpec((1, 1, blk), lambda i, j: (j, 0, 0)),
            pl.BlockSpec((blk, E), lambda i, j: (i, 0)),
            pl.BlockSpec((blk, E), lambda i, j: (j, 0)),
            pl.BlockSpec((B, blk, D), lambda i, j: (0, j, 0)),
            pl.BlockSpec((B, blk, D), lambda i, j: (0, i, 0)),
        ],
        out_specs=pl.BlockSpec((B, blk, D), lambda i, j: (0, i, 0)),
        out_shape=jax.ShapeDtypeStruct((B, L, D), x.dtype),
        scratch_shapes=[
            pltpu.VMEM((B, blk, D), jnp.float32),
            pltpu.VMEM((blk, 1), jnp.float32),
        ],
        compiler_params=pltpu.CompilerParams(
            dimension_semantics=("arbitrary", "arbitrary")),
    )(mask_col, mask_row, emb_i_weight, emb_j_weight, x, x)
